# baseline (speedup 1.0000x reference)
import jax
import jax.numpy as jnp
from jax import lax
from jax.experimental import pallas as pl
from jax.experimental.pallas import tpu as pltpu
from jax.experimental.pallas import tpu_sc as plsc

N = 10000
E = 320000
D = 128
DH = D // 2

NC, NS = 2, 16
EP = 327680
NP = 10240
BI = 128
RW = EP // BI // NS
SCH = 128
RPS = SCH // BI
NCH = RW // RPS
RPT = NP // NS


EPH = EP // 2
BEH = 2048


def _pe_body(ew_ref, w1_ref, w2_ref, out_ref):
    hp = lax.dot_general(ew_ref[...], w1_ref[...],
                         (((0,), (0,)), ((), ())),
                         preferred_element_type=jnp.float32)
    hp = jnp.maximum(hp, 0.0).astype(jnp.bfloat16)
    out_ref[0] = (
        jnp.dot(hp, w2_ref[0], preferred_element_type=jnp.float32) + 1.0
    ).astype(jnp.bfloat16)


def _pe_call(ew4t, w1p, w2d):
    return pl.pallas_call(
        _pe_body,
        grid=(EPH // BEH, NC),
        in_specs=[
            pl.BlockSpec((4, BEH), lambda i, h: (0, i)),
            pl.BlockSpec((4, D), lambda i, h: (0, 0)),
            pl.BlockSpec((1, D, D), lambda i, h: (h, 0, 0)),
        ],
        out_specs=pl.BlockSpec((1, BEH, D), lambda i, h: (h, i, 0)),
        out_shape=jax.ShapeDtypeStruct((NC, EPH, D), jnp.bfloat16),
    )(ew4t, w1p, w2d)


def _sc_body(x_hbm, src_hbm, dst_hbm, pe_hbm, acc_hbm, cnt_hbm,
             src_a, src_b, dst_v, pe_a, pe_b, xr_a, xr_b, ms_a, ms_b,
             cnt_v, acc_sh, semg_a, semg_b, semi_a, semi_b, sems_a, sems_b):
    c = lax.axis_index("c")
    s = lax.axis_index("s")

    zeros16 = jnp.zeros((16,), jnp.float32)
    ones16 = jnp.ones((16,), jnp.float32)

    @pl.loop(0, NP // 16)
    def _(i):
        cnt_v[pl.ds(i * 16, 16)] = zeros16

    @pl.loop(0, SCH)
    def _(r):
        for g in range(DH // 16):
            ms_a[r, pl.ds(g * 16, 16)] = zeros16

    for q in range(RPT // SCH):
        pltpu.sync_copy(ms_a, acc_sh.at[pl.ds(s * RPT + q * SCH, SCH)])

    pltpu.sync_copy(dst_hbm.at[pl.ds(s * RW, RW)], dst_v)
    plsc.subcore_barrier()

    shalf = s // 8
    coff = shalf * DH

    def idx_fire(k, src_buf, sem):
        pltpu.async_copy(src_hbm.at[pl.ds(s * RW + k, 1)], src_buf, sem)

    def idx_wait_gidx(src_buf, sem):
        pltpu.make_async_copy(src_hbm.at[pl.ds(0, 1)], src_buf, sem).wait()
        for g in range(BI // 16):
            sl = pl.ds(g * 16, 16)
            src_buf[0, sl] = src_buf[0, sl] + c * N

    def fire(k, src_buf, pe_buf, xr_buf, sem):
        pltpu.async_copy(x_hbm.at[src_buf.at[0]], xr_buf, sem)
        prow = (s * RW + k) * BI - shalf * EPH
        pltpu.async_copy(pe_hbm.at[c, pl.ds(prow, SCH), pl.ds(coff, DH)],
                         pe_buf, sem)

    def drain(src_buf, pe_buf, xr_buf, sem):
        pltpu.make_async_copy(x_hbm.at[src_buf.at[0]], xr_buf, sem).wait()
        pltpu.make_async_copy(pe_hbm.at[c, pl.ds(0, SCH), pl.ds(0, DH)],
                              pe_buf, sem).wait()

    def scatter_wait(ms_buf, sem):
        pltpu.make_async_copy(ms_buf, acc_sh.at[dst_v.at[0]], sem).wait()

    def chunk_step(kk, k, src_buf, pe_buf, xr_buf, ms_buf, semg, semi, sems):
        drain(src_buf, pe_buf, xr_buf, semg)

        @pl.when(k + 2 < NCH)
        def _():
            idx_fire(k + 2, src_buf, semi)

        @pl.when(kk > 0)
        def _():
            scatter_wait(ms_buf, sems)

        @plsc.parallel_loop(0, SCH, 1, unroll=4)
        def _(r):
            for g in range(DH // 32):
                sl = pl.ds(g * 32, 32)
                pa, pb = plsc.unpack(pe_buf[r, sl],
                                     format=plsc.PackFormat.INTERLEAVED)
                xa, xb = plsc.unpack(xr_buf[r, sl],
                                     format=plsc.PackFormat.INTERLEAVED)
                ms_buf[r, pl.ds(g * 32, 16)] = xa * pa
                ms_buf[r, pl.ds(g * 32 + 16, 16)] = xb * pb

        pltpu.async_copy(ms_buf, acc_sh.at[dst_v.at[k]], sems, add=True)

        @pl.when((c == 0) == (k < NCH // 2))
        def _():
            for g in range(BI // 16):
                idx16 = dst_v[k, pl.ds(g * 16, 16)]
                plsc.addupdate_scatter(cnt_v, [idx16], ones16)

        @pl.when(k + 2 < NCH)
        def _():
            idx_wait_gidx(src_buf, semi)
            fire(k + 2, src_buf, pe_buf, xr_buf, semg)

    pltpu.sync_copy(src_hbm.at[pl.ds(s * RW, 1)], src_a)
    pltpu.sync_copy(src_hbm.at[pl.ds(s * RW + 1, 1)], src_b)
    for g in range(BI // 16):
        sl = pl.ds(g * 16, 16)
        src_a[0, sl] = src_a[0, sl] + c * N
        src_b[0, sl] = src_b[0, sl] + c * N
    fire(0, src_a, pe_a, xr_a, semg_a)
    fire(1, src_b, pe_b, xr_b, semg_b)

    @pl.loop(0, NCH // 2)
    def _(kk):
        ka = 2 * kk
        chunk_step(kk, ka, src_a, pe_a, xr_a, ms_a, semg_a, semi_a, sems_a)
        chunk_step(kk, ka + 1, src_b, pe_b, xr_b, ms_b, semg_b, semi_b,
                   sems_b)

    scatter_wait(ms_a, sems_a)
    scatter_wait(ms_b, sems_b)
    plsc.subcore_barrier()

    pltpu.sync_copy(acc_sh.at[pl.ds(s * RPT, RPT)],
                    acc_hbm.at[c, pl.ds(s * RPT, RPT)])
    pltpu.sync_copy(cnt_v, cnt_hbm.at[pl.ds((c * NS + s) * NP, NP)])


_sc_call = pl.kernel(
    _sc_body,
    out_type=[
        jax.ShapeDtypeStruct((NC, NP, DH), jnp.float32),
        jax.ShapeDtypeStruct((NC * NS * NP,), jnp.float32),
    ],
    mesh=plsc.VectorSubcoreMesh(core_axis_name="c", subcore_axis_name="s"),
    compiler_params=pltpu.CompilerParams(needs_layout_passes=False,
                                         use_tc_tiling_on_sc=False),
    scratch_types=[
        pltpu.VMEM((1, BI), jnp.int32),
        pltpu.VMEM((1, BI), jnp.int32),
        pltpu.VMEM((RW, BI), jnp.int32),
        pltpu.VMEM((SCH, DH), jnp.bfloat16),
        pltpu.VMEM((SCH, DH), jnp.bfloat16),
        pltpu.VMEM((SCH, DH), jnp.bfloat16),
        pltpu.VMEM((SCH, DH), jnp.bfloat16),
        pltpu.VMEM((SCH, DH), jnp.float32),
        pltpu.VMEM((SCH, DH), jnp.float32),
        pltpu.VMEM((NP,), jnp.float32),
        pltpu.VMEM_SHARED((NP, DH), jnp.float32),
        pltpu.SemaphoreType.DMA,
        pltpu.SemaphoreType.DMA,
        pltpu.SemaphoreType.DMA,
        pltpu.SemaphoreType.DMA,
        pltpu.SemaphoreType.DMA,
        pltpu.SemaphoreType.DMA,
    ],
)


BN = 2000
NB = N // BN


def _fin_body(acc_ref, cnt_ref, x_ref, w_ref, b_ref, g_ref, be_ref,
              out_ref, t_sc, s1_sc, s2_sc):
    p = pl.program_id(0)
    i = pl.program_id(1)

    @pl.when(p == 0)
    def _():
        xb = x_ref[...]
        ssum = jnp.concatenate([acc_ref[0], acc_ref[1]], axis=1) + xb
        cnt = jnp.sum(cnt_ref[...], axis=1) + 1.0
        mean = ssum / cnt[:, None]
        wt = w_ref[...]
        t_pre = (
            jnp.dot(xb, wt[:D], preferred_element_type=jnp.float32)
            + jnp.dot(mean, wt[D:], preferred_element_type=jnp.float32)
            + b_ref[...][None, :]
        )
        nrm = jnp.sqrt(jnp.sum(t_pre * t_pre, axis=1, keepdims=True))
        t = t_pre / jnp.maximum(nrm, 1e-12)
        t_sc[pl.ds(i * BN, BN), :] = t

        @pl.when(i == 0)
        def _():
            s1_sc[...] = jnp.zeros_like(s1_sc)
            s2_sc[...] = jnp.zeros_like(s2_sc)

        s1_sc[...] += jnp.sum(t, axis=0, keepdims=True)
        s2_sc[...] += jnp.sum(t * t, axis=0, keepdims=True)

    @pl.when(p == 1)
    def _():
        t = t_sc[pl.ds(i * BN, BN), :]
        mu = s1_sc[...] / N
        var = s2_sc[...] / N - mu * mu
        y = (t - mu) * lax.rsqrt(var + 1e-5) * g_ref[...][None, :] \
            + be_ref[...][None, :]
        out_ref[...] = jnp.maximum(y + x_ref[...], 0.0)


def _fin_call(acc, cntp, x, W, b, gamma, beta):
    return pl.pallas_call(
        _fin_body,
        grid=(2, NB),
        in_specs=[
            pl.BlockSpec((NC, BN, DH), lambda p, i: (0, i, 0)),
            pl.BlockSpec((BN, NC * NS), lambda p, i: (i, 0)),
            pl.BlockSpec((BN, D), lambda p, i: (i, 0)),
            pl.BlockSpec((2 * D, D), lambda p, i: (0, 0)),
            pl.BlockSpec((D,), lambda p, i: (0,)),
            pl.BlockSpec((D,), lambda p, i: (0,)),
            pl.BlockSpec((D,), lambda p, i: (0,)),
        ],
        out_specs=pl.BlockSpec((BN, D), lambda p, i: (i, 0)),
        out_shape=jax.ShapeDtypeStruct((N, D), jnp.float32),
        scratch_shapes=[
            pltpu.VMEM((N, D), jnp.float32),
            pltpu.VMEM((1, D), jnp.float32),
            pltpu.VMEM((1, D), jnp.float32),
        ],
    )(acc, cntp, x, W, b, gamma, beta)


def kernel(x, edge_index, edge_w, W1, W2, W, b, gamma, beta):
    pad = EP - E
    src2d = jnp.concatenate(
        [edge_index[0], jnp.zeros((pad,), jnp.int32)]).reshape(EP // BI, BI)
    dst2d = jnp.concatenate(
        [edge_index[1], jnp.full((pad,), N, jnp.int32)]).reshape(EP // BI, BI)
    ewt = jnp.concatenate(
        [edge_w.T, jnp.zeros((2, pad), jnp.float32)], axis=1)
    ew4t = jnp.concatenate([ewt[:, :EPH], ewt[:, EPH:]], axis=0)
    q = jnp.arange(DH)
    perm = 32 * (q // 32) + (q % 32) // 2 + 16 * (q % 2)
    xcat = jnp.concatenate(
        [x[:, :DH], x[:, DH:]], axis=0)[:, perm].astype(jnp.bfloat16)
    z2 = jnp.zeros((2, DH), jnp.float32)
    w1p = jnp.concatenate(
        [jnp.concatenate([W1, z2], axis=1),
         jnp.concatenate([z2, W1], axis=1)], axis=0)
    zd = jnp.zeros((DH, DH), jnp.float32)
    w2d = jnp.stack([
        jnp.concatenate(
            [jnp.concatenate([W2[:, c * DH:(c + 1) * DH], zd], axis=1),
             jnp.concatenate([zd, W2[:, c * DH:(c + 1) * DH]], axis=1)],
            axis=0)
        for c in range(NC)])
    qd = jnp.arange(D)
    permd = 32 * (qd // 32) + (qd % 32) // 2 + 16 * (qd % 2)
    w2d = w2d[:, :, permd].astype(jnp.bfloat16)
    pe1 = _pe_call(ew4t, w1p, w2d)
    acc, cntp = _sc_call(xcat, src2d, dst2d, pe1)
    return _fin_call(acc, cntp.reshape(NC * NS, NP).T, x, W, b, gamma, beta)


# speedup vs baseline: 1.3819x; 1.3819x over previous
import jax
import jax.numpy as jnp
from jax import lax
from jax.experimental import pallas as pl
from jax.experimental.pallas import tpu as pltpu
from jax.experimental.pallas import tpu_sc as plsc

N = 10000
E = 320000
D = 128
DH = D // 2

NC, NS = 2, 16
EP = 327680
NP = 10240
BI = 128
RW = EP // BI // NS
SCH = 128
RPS = SCH // BI
NCH = RW // RPS
RPT = NP // NS


EPH = EP // 2
BEH = 2048


def _pe_body(ew_ref, w1_ref, w2_ref, out_ref):
    hp = lax.dot_general(ew_ref[...], w1_ref[...],
                         (((0,), (0,)), ((), ())),
                         preferred_element_type=jnp.float32)
    hp = jnp.maximum(hp, 0.0).astype(jnp.bfloat16)
    out_ref[0] = (
        jnp.dot(hp, w2_ref[0], preferred_element_type=jnp.float32) + 1.0
    )


def _pe_call(ew4t, w1p, w2d):
    return pl.pallas_call(
        _pe_body,
        grid=(EPH // BEH, NC),
        in_specs=[
            pl.BlockSpec((4, BEH), lambda i, h: (0, i)),
            pl.BlockSpec((4, D), lambda i, h: (0, 0)),
            pl.BlockSpec((1, D, D), lambda i, h: (h, 0, 0)),
        ],
        out_specs=pl.BlockSpec((1, BEH, D), lambda i, h: (h, i, 0)),
        out_shape=jax.ShapeDtypeStruct((NC, EPH, D), jnp.float32),
    )(ew4t, w1p, w2d)


def _sc_body(x_hbm, src_hbm, dst_hbm, pe_hbm, acc_hbm, cnt_hbm,
             src_a, src_b, dst_v, pe_a, pe_b, xr_a, xr_b, ms_a, ms_b,
             cnt_v, acc_sh, semg_a, semg_b, semi_a, semi_b, sems_a, sems_b):
    c = lax.axis_index("c")
    s = lax.axis_index("s")

    zeros16 = jnp.zeros((16,), jnp.float32)
    ones16 = jnp.ones((16,), jnp.float32)

    @pl.loop(0, NP // 16)
    def _(i):
        cnt_v[pl.ds(i * 16, 16)] = zeros16

    @pl.loop(0, SCH)
    def _(r):
        for g in range(DH // 16):
            ms_a[r, pl.ds(g * 16, 16)] = zeros16

    for q in range(RPT // SCH):
        pltpu.sync_copy(ms_a, acc_sh.at[pl.ds(s * RPT + q * SCH, SCH)])

    pltpu.sync_copy(dst_hbm.at[pl.ds(s * RW, RW)], dst_v)
    plsc.subcore_barrier()

    shalf = s // 8
    coff = shalf * DH

    def idx_fire(k, src_buf, sem):
        pltpu.async_copy(src_hbm.at[pl.ds(s * RW + k, 1)], src_buf, sem)

    def idx_wait_gidx(src_buf, sem):
        pltpu.make_async_copy(src_hbm.at[pl.ds(0, 1)], src_buf, sem).wait()
        for g in range(BI // 16):
            sl = pl.ds(g * 16, 16)
            src_buf[0, sl] = src_buf[0, sl] + c * N

    def fire(k, src_buf, pe_buf, xr_buf, sem):
        pltpu.async_copy(x_hbm.at[src_buf.at[0]], xr_buf, sem)
        prow = (s * RW + k) * BI - shalf * EPH
        pltpu.async_copy(pe_hbm.at[c, pl.ds(prow, SCH), pl.ds(coff, DH)],
                         pe_buf, sem)

    def drain(src_buf, pe_buf, xr_buf, sem):
        pltpu.make_async_copy(x_hbm.at[src_buf.at[0]], xr_buf, sem).wait()
        pltpu.make_async_copy(pe_hbm.at[c, pl.ds(0, SCH), pl.ds(0, DH)],
                              pe_buf, sem).wait()

    def scatter_wait(ms_buf, sem):
        pltpu.make_async_copy(ms_buf, acc_sh.at[dst_v.at[0]], sem).wait()

    def chunk_step(kk, k, src_buf, pe_buf, xr_buf, ms_buf, semg, semi, sems):
        drain(src_buf, pe_buf, xr_buf, semg)

        @pl.when(k + 2 < NCH)
        def _():
            idx_fire(k + 2, src_buf, semi)

        @pl.when(kk > 0)
        def _():
            scatter_wait(ms_buf, sems)

        @plsc.parallel_loop(0, SCH, 1, unroll=4)
        def _(r):
            for g in range(DH // 32):
                xa, xb = plsc.unpack(xr_buf[r, pl.ds(g * 32, 32)],
                                     format=plsc.PackFormat.INTERLEAVED)
                ms_buf[r, pl.ds(g * 32, 16)] = \
                    xa * pe_buf[r, pl.ds(g * 32, 16)]
                ms_buf[r, pl.ds(g * 32 + 16, 16)] = \
                    xb * pe_buf[r, pl.ds(g * 32 + 16, 16)]

        pltpu.async_copy(ms_buf, acc_sh.at[dst_v.at[k]], sems, add=True)

        @pl.when((c == 0) == (k < NCH // 2))
        def _():
            for g in range(BI // 16):
                idx16 = dst_v[k, pl.ds(g * 16, 16)]
                plsc.addupdate_scatter(cnt_v, [idx16], ones16)

        @pl.when(k + 2 < NCH)
        def _():
            idx_wait_gidx(src_buf, semi)
            fire(k + 2, src_buf, pe_buf, xr_buf, semg)

    pltpu.sync_copy(src_hbm.at[pl.ds(s * RW, 1)], src_a)
    pltpu.sync_copy(src_hbm.at[pl.ds(s * RW + 1, 1)], src_b)
    for g in range(BI // 16):
        sl = pl.ds(g * 16, 16)
        src_a[0, sl] = src_a[0, sl] + c * N
        src_b[0, sl] = src_b[0, sl] + c * N
    fire(0, src_a, pe_a, xr_a, semg_a)
    fire(1, src_b, pe_b, xr_b, semg_b)

    @pl.loop(0, NCH // 2)
    def _(kk):
        ka = 2 * kk
        chunk_step(kk, ka, src_a, pe_a, xr_a, ms_a, semg_a, semi_a, sems_a)
        chunk_step(kk, ka + 1, src_b, pe_b, xr_b, ms_b, semg_b, semi_b,
                   sems_b)

    scatter_wait(ms_a, sems_a)
    scatter_wait(ms_b, sems_b)
    plsc.subcore_barrier()

    pltpu.sync_copy(acc_sh.at[pl.ds(s * RPT, RPT)],
                    acc_hbm.at[c, pl.ds(s * RPT, RPT)])
    pltpu.sync_copy(cnt_v, cnt_hbm.at[pl.ds((c * NS + s) * NP, NP)])


_sc_call = pl.kernel(
    _sc_body,
    out_type=[
        jax.ShapeDtypeStruct((NC, NP, DH), jnp.float32),
        jax.ShapeDtypeStruct((NC * NS * NP,), jnp.float32),
    ],
    mesh=plsc.VectorSubcoreMesh(core_axis_name="c", subcore_axis_name="s"),
    compiler_params=pltpu.CompilerParams(needs_layout_passes=False,
                                         use_tc_tiling_on_sc=False),
    scratch_types=[
        pltpu.VMEM((1, BI), jnp.int32),
        pltpu.VMEM((1, BI), jnp.int32),
        pltpu.VMEM((RW, BI), jnp.int32),
        pltpu.VMEM((SCH, DH), jnp.float32),
        pltpu.VMEM((SCH, DH), jnp.float32),
        pltpu.VMEM((SCH, DH), jnp.bfloat16),
        pltpu.VMEM((SCH, DH), jnp.bfloat16),
        pltpu.VMEM((SCH, DH), jnp.float32),
        pltpu.VMEM((SCH, DH), jnp.float32),
        pltpu.VMEM((NP,), jnp.float32),
        pltpu.VMEM_SHARED((NP, DH), jnp.float32),
        pltpu.SemaphoreType.DMA,
        pltpu.SemaphoreType.DMA,
        pltpu.SemaphoreType.DMA,
        pltpu.SemaphoreType.DMA,
        pltpu.SemaphoreType.DMA,
        pltpu.SemaphoreType.DMA,
    ],
)


BN = 2000
NB = N // BN


def _fin_body(acc_ref, cnt_ref, x_ref, w_ref, b_ref, g_ref, be_ref,
              out_ref, t_sc, s1_sc, s2_sc):
    p = pl.program_id(0)
    i = pl.program_id(1)

    @pl.when(p == 0)
    def _():
        xb = x_ref[...]
        ssum = jnp.concatenate([acc_ref[0], acc_ref[1]], axis=1) + xb
        cnt = jnp.sum(cnt_ref[...], axis=1) + 1.0
        mean = ssum / cnt[:, None]
        wt = w_ref[...]
        t_pre = (
            jnp.dot(xb, wt[:D], preferred_element_type=jnp.float32)
            + jnp.dot(mean, wt[D:], preferred_element_type=jnp.float32)
            + b_ref[...][None, :]
        )
        nrm = jnp.sqrt(jnp.sum(t_pre * t_pre, axis=1, keepdims=True))
        t = t_pre / jnp.maximum(nrm, 1e-12)
        t_sc[pl.ds(i * BN, BN), :] = t

        @pl.when(i == 0)
        def _():
            s1_sc[...] = jnp.zeros_like(s1_sc)
            s2_sc[...] = jnp.zeros_like(s2_sc)

        s1_sc[...] += jnp.sum(t, axis=0, keepdims=True)
        s2_sc[...] += jnp.sum(t * t, axis=0, keepdims=True)

    @pl.when(p == 1)
    def _():
        t = t_sc[pl.ds(i * BN, BN), :]
        mu = s1_sc[...] / N
        var = s2_sc[...] / N - mu * mu
        y = (t - mu) * lax.rsqrt(var + 1e-5) * g_ref[...][None, :] \
            + be_ref[...][None, :]
        out_ref[...] = jnp.maximum(y + x_ref[...], 0.0)


def _fin_call(acc, cntp, x, W, b, gamma, beta):
    return pl.pallas_call(
        _fin_body,
        grid=(2, NB),
        in_specs=[
            pl.BlockSpec((NC, BN, DH), lambda p, i: (0, i, 0)),
            pl.BlockSpec((BN, NC * NS), lambda p, i: (i, 0)),
            pl.BlockSpec((BN, D), lambda p, i: (i, 0)),
            pl.BlockSpec((2 * D, D), lambda p, i: (0, 0)),
            pl.BlockSpec((D,), lambda p, i: (0,)),
            pl.BlockSpec((D,), lambda p, i: (0,)),
            pl.BlockSpec((D,), lambda p, i: (0,)),
        ],
        out_specs=pl.BlockSpec((BN, D), lambda p, i: (i, 0)),
        out_shape=jax.ShapeDtypeStruct((N, D), jnp.float32),
        scratch_shapes=[
            pltpu.VMEM((N, D), jnp.float32),
            pltpu.VMEM((1, D), jnp.float32),
            pltpu.VMEM((1, D), jnp.float32),
        ],
    )(acc, cntp, x, W, b, gamma, beta)


def kernel(x, edge_index, edge_w, W1, W2, W, b, gamma, beta):
    pad = EP - E
    src2d = jnp.concatenate(
        [edge_index[0], jnp.zeros((pad,), jnp.int32)]).reshape(EP // BI, BI)
    dst2d = jnp.concatenate(
        [edge_index[1], jnp.full((pad,), N, jnp.int32)]).reshape(EP // BI, BI)
    ewt = jnp.concatenate(
        [edge_w.T, jnp.zeros((2, pad), jnp.float32)], axis=1)
    ew4t = jnp.concatenate([ewt[:, :EPH], ewt[:, EPH:]], axis=0)
    xcat = jnp.concatenate([x[:, :DH], x[:, DH:]], axis=0)
    xcat = (xcat.reshape(2 * N, 2, 2, 16).transpose(0, 1, 3, 2)
            .reshape(2 * N, DH).astype(jnp.bfloat16))
    z2 = jnp.zeros((2, DH), jnp.float32)
    w1p = jnp.concatenate(
        [jnp.concatenate([W1, z2], axis=1),
         jnp.concatenate([z2, W1], axis=1)], axis=0)
    zd = jnp.zeros((DH, DH), jnp.float32)
    w2d = jnp.stack([
        jnp.concatenate(
            [jnp.concatenate([W2[:, c * DH:(c + 1) * DH], zd], axis=1),
             jnp.concatenate([zd, W2[:, c * DH:(c + 1) * DH]], axis=1)],
            axis=0)
        for c in range(NC)])
    pe1 = _pe_call(ew4t, w1p, w2d.astype(jnp.bfloat16))
    acc, cntp = _sc_call(xcat, src2d, dst2d, pe1)
    return _fin_call(acc, cntp.reshape(NC * NS, NP).T, x, W, b, gamma, beta)


# speedup vs baseline: 1.4811x; 1.0718x over previous
import jax
import jax.numpy as jnp
from jax import lax
from jax.experimental import pallas as pl
from jax.experimental.pallas import tpu as pltpu
from jax.experimental.pallas import tpu_sc as plsc

N = 10000
E = 320000
D = 128
DH = D // 2

NC, NS = 2, 16
EP = 327680
NP = 10240
BI = 128
RW = EP // BI // NS
SCH = 128
RPS = SCH // BI
NCH = RW // RPS
RPT = NP // NS


EPH = EP // 2
BEH = 2048


def _pe_body(ew_ref, w1_ref, w2_ref, out_ref):
    hp = lax.dot_general(ew_ref[...], w1_ref[...],
                         (((0,), (0,)), ((), ())),
                         preferred_element_type=jnp.float32)
    hp = jnp.maximum(hp, 0.0).astype(jnp.bfloat16)
    pe = jnp.dot(hp, w2_ref[0], preferred_element_type=jnp.float32) + 1.0
    rb = pe.astype(jnp.bfloat16).astype(jnp.float32)
    u = lax.bitcast_convert_type(rb, jnp.uint32).reshape(BEH // 128, 2,
                                                         64, D)
    packed = u[:, 0] | (u[:, 1] >> 16)
    out_ref[0] = lax.bitcast_convert_type(
        packed, jnp.int32).reshape(BEH // 2, D)


def _pe_call(ew4t, w1p, w2d):
    return pl.pallas_call(
        _pe_body,
        grid=(EPH // BEH, NC),
        in_specs=[
            pl.BlockSpec((4, BEH), lambda i, h: (0, i)),
            pl.BlockSpec((4, D), lambda i, h: (0, 0)),
            pl.BlockSpec((1, D, D), lambda i, h: (h, 0, 0)),
        ],
        out_specs=pl.BlockSpec((1, BEH // 2, D), lambda i, h: (h, i, 0)),
        out_shape=jax.ShapeDtypeStruct((NC, EPH // 2, D), jnp.int32),
    )(ew4t, w1p, w2d)


def _sc_body(x_hbm, src_hbm, dst_hbm, pe_hbm, acc_hbm, cnt_hbm,
             src_a, src_b, dst_v, pe_a, pe_b, xr_a, xr_b, ms_a, ms_b,
             cnt_v, acc_sh, semg_a, semg_b, semi_a, semi_b, sems_a, sems_b):
    c = lax.axis_index("c")
    s = lax.axis_index("s")

    zeros16 = jnp.zeros((16,), jnp.float32)
    ones16 = jnp.ones((16,), jnp.float32)

    @pl.loop(0, NP // 16)
    def _(i):
        cnt_v[pl.ds(i * 16, 16)] = zeros16

    @pl.loop(0, SCH)
    def _(r):
        for g in range(DH // 16):
            ms_a[r, pl.ds(g * 16, 16)] = zeros16

    for q in range(RPT // SCH):
        pltpu.sync_copy(ms_a, acc_sh.at[pl.ds(s * RPT + q * SCH, SCH)])

    pltpu.sync_copy(dst_hbm.at[pl.ds(s * RW, RW)], dst_v)
    plsc.subcore_barrier()

    shalf = s // 8
    coff = shalf * DH

    def idx_fire(k, src_buf, sem):
        pltpu.async_copy(src_hbm.at[pl.ds(s * RW + k, 1)], src_buf, sem)

    def idx_wait_gidx(src_buf, sem):
        pltpu.make_async_copy(src_hbm.at[pl.ds(0, 1)], src_buf, sem).wait()
        for g in range(BI // 16):
            sl = pl.ds(g * 16, 16)
            src_buf[0, sl] = src_buf[0, sl] + c * N

    def fire(k, src_buf, pe_buf, xr_buf, sem):
        pltpu.async_copy(x_hbm.at[src_buf.at[0]], xr_buf, sem)
        prow = ((s * RW + k) * BI - shalf * EPH) // 2
        pltpu.async_copy(
            pe_hbm.at[c, pl.ds(prow, SCH // 2), pl.ds(coff, DH)],
            pe_buf, sem)

    def drain(src_buf, pe_buf, xr_buf, sem):
        pltpu.make_async_copy(x_hbm.at[src_buf.at[0]], xr_buf, sem).wait()
        pltpu.make_async_copy(
            pe_hbm.at[c, pl.ds(0, SCH // 2), pl.ds(0, DH)],
            pe_buf, sem).wait()

    def scatter_wait(ms_buf, sem):
        pltpu.make_async_copy(ms_buf, acc_sh.at[dst_v.at[0]], sem).wait()

    def chunk_step(kk, k, src_buf, pe_buf, xr_buf, ms_buf, semg, semi, sems):
        drain(src_buf, pe_buf, xr_buf, semg)

        @pl.when(k + 2 < NCH)
        def _():
            idx_fire(k + 2, src_buf, semi)

        @pl.when(kk > 0)
        def _():
            scatter_wait(ms_buf, sems)

        hmask = jnp.int32(-65536)

        @plsc.parallel_loop(0, SCH // 2, 1, unroll=2)
        def _(rp):
            for gg in range(DH // 32):
                w0 = pe_buf[rp, pl.ds(gg * 32, 16)]
                w1 = pe_buf[rp, pl.ds(gg * 32 + 16, 16)]
                xa, xb = plsc.unpack(xr_buf[rp, pl.ds(gg * 32, 32)],
                                     format=plsc.PackFormat.INTERLEAVED)
                ms_buf[rp, pl.ds(gg * 32, 16)] = \
                    xa * plsc.bitcast(w0 & hmask, jnp.float32)
                ms_buf[rp, pl.ds(gg * 32 + 16, 16)] = \
                    xb * plsc.bitcast(w1 & hmask, jnp.float32)
                r2 = rp + SCH // 2
                xc, xd = plsc.unpack(xr_buf[r2, pl.ds(gg * 32, 32)],
                                     format=plsc.PackFormat.INTERLEAVED)
                ms_buf[r2, pl.ds(gg * 32, 16)] = \
                    xc * plsc.bitcast(w0 << 16, jnp.float32)
                ms_buf[r2, pl.ds(gg * 32 + 16, 16)] = \
                    xd * plsc.bitcast(w1 << 16, jnp.float32)

        pltpu.async_copy(ms_buf, acc_sh.at[dst_v.at[k]], sems, add=True)

        @pl.when((c == 0) == (k < NCH // 2))
        def _():
            for g in range(BI // 16):
                idx16 = dst_v[k, pl.ds(g * 16, 16)]
                plsc.addupdate_scatter(cnt_v, [idx16], ones16)

        @pl.when(k + 2 < NCH)
        def _():
            idx_wait_gidx(src_buf, semi)
            fire(k + 2, src_buf, pe_buf, xr_buf, semg)

    pltpu.sync_copy(src_hbm.at[pl.ds(s * RW, 1)], src_a)
    pltpu.sync_copy(src_hbm.at[pl.ds(s * RW + 1, 1)], src_b)
    for g in range(BI // 16):
        sl = pl.ds(g * 16, 16)
        src_a[0, sl] = src_a[0, sl] + c * N
        src_b[0, sl] = src_b[0, sl] + c * N
    fire(0, src_a, pe_a, xr_a, semg_a)
    fire(1, src_b, pe_b, xr_b, semg_b)

    @pl.loop(0, NCH // 2)
    def _(kk):
        ka = 2 * kk
        chunk_step(kk, ka, src_a, pe_a, xr_a, ms_a, semg_a, semi_a, sems_a)
        chunk_step(kk, ka + 1, src_b, pe_b, xr_b, ms_b, semg_b, semi_b,
                   sems_b)

    scatter_wait(ms_a, sems_a)
    scatter_wait(ms_b, sems_b)
    plsc.subcore_barrier()

    pltpu.sync_copy(acc_sh.at[pl.ds(s * RPT, RPT)],
                    acc_hbm.at[c, pl.ds(s * RPT, RPT)])
    pltpu.sync_copy(cnt_v, cnt_hbm.at[pl.ds((c * NS + s) * NP, NP)])


_sc_call = pl.kernel(
    _sc_body,
    out_type=[
        jax.ShapeDtypeStruct((NC, NP, DH), jnp.float32),
        jax.ShapeDtypeStruct((NC * NS * NP,), jnp.float32),
    ],
    mesh=plsc.VectorSubcoreMesh(core_axis_name="c", subcore_axis_name="s"),
    compiler_params=pltpu.CompilerParams(needs_layout_passes=False,
                                         use_tc_tiling_on_sc=False),
    scratch_types=[
        pltpu.VMEM((1, BI), jnp.int32),
        pltpu.VMEM((1, BI), jnp.int32),
        pltpu.VMEM((RW, BI), jnp.int32),
        pltpu.VMEM((SCH // 2, DH), jnp.int32),
        pltpu.VMEM((SCH // 2, DH), jnp.int32),
        pltpu.VMEM((SCH, DH), jnp.bfloat16),
        pltpu.VMEM((SCH, DH), jnp.bfloat16),
        pltpu.VMEM((SCH, DH), jnp.float32),
        pltpu.VMEM((SCH, DH), jnp.float32),
        pltpu.VMEM((NP,), jnp.float32),
        pltpu.VMEM_SHARED((NP, DH), jnp.float32),
        pltpu.SemaphoreType.DMA,
        pltpu.SemaphoreType.DMA,
        pltpu.SemaphoreType.DMA,
        pltpu.SemaphoreType.DMA,
        pltpu.SemaphoreType.DMA,
        pltpu.SemaphoreType.DMA,
    ],
)


BN = 2000
NB = N // BN


def _fin_body(acc_ref, cnt_ref, x_ref, w_ref, b_ref, g_ref, be_ref,
              out_ref, t_sc, s1_sc, s2_sc):
    p = pl.program_id(0)
    i = pl.program_id(1)

    @pl.when(p == 0)
    def _():
        xb = x_ref[...]
        ssum = jnp.concatenate([acc_ref[0], acc_ref[1]], axis=1) + xb
        cnt = jnp.sum(cnt_ref[...], axis=1) + 1.0
        mean = ssum / cnt[:, None]
        wt = w_ref[...]
        t_pre = (
            jnp.dot(xb, wt[:D], preferred_element_type=jnp.float32)
            + jnp.dot(mean, wt[D:], preferred_element_type=jnp.float32)
            + b_ref[...][None, :]
        )
        nrm = jnp.sqrt(jnp.sum(t_pre * t_pre, axis=1, keepdims=True))
        t = t_pre / jnp.maximum(nrm, 1e-12)
        t_sc[pl.ds(i * BN, BN), :] = t

        @pl.when(i == 0)
        def _():
            s1_sc[...] = jnp.zeros_like(s1_sc)
            s2_sc[...] = jnp.zeros_like(s2_sc)

        s1_sc[...] += jnp.sum(t, axis=0, keepdims=True)
        s2_sc[...] += jnp.sum(t * t, axis=0, keepdims=True)

    @pl.when(p == 1)
    def _():
        t = t_sc[pl.ds(i * BN, BN), :]
        mu = s1_sc[...] / N
        var = s2_sc[...] / N - mu * mu
        y = (t - mu) * lax.rsqrt(var + 1e-5) * g_ref[...][None, :] \
            + be_ref[...][None, :]
        out_ref[...] = jnp.maximum(y + x_ref[...], 0.0)


def _fin_call(acc, cntp, x, W, b, gamma, beta):
    return pl.pallas_call(
        _fin_body,
        grid=(2, NB),
        in_specs=[
            pl.BlockSpec((NC, BN, DH), lambda p, i: (0, i, 0)),
            pl.BlockSpec((BN, NC * NS), lambda p, i: (i, 0)),
            pl.BlockSpec((BN, D), lambda p, i: (i, 0)),
            pl.BlockSpec((2 * D, D), lambda p, i: (0, 0)),
            pl.BlockSpec((D,), lambda p, i: (0,)),
            pl.BlockSpec((D,), lambda p, i: (0,)),
            pl.BlockSpec((D,), lambda p, i: (0,)),
        ],
        out_specs=pl.BlockSpec((BN, D), lambda p, i: (i, 0)),
        out_shape=jax.ShapeDtypeStruct((N, D), jnp.float32),
        scratch_shapes=[
            pltpu.VMEM((N, D), jnp.float32),
            pltpu.VMEM((1, D), jnp.float32),
            pltpu.VMEM((1, D), jnp.float32),
        ],
    )(acc, cntp, x, W, b, gamma, beta)


def kernel(x, edge_index, edge_w, W1, W2, W, b, gamma, beta):
    pad = EP - E
    src2d = jnp.concatenate(
        [edge_index[0], jnp.zeros((pad,), jnp.int32)]).reshape(EP // BI, BI)
    dst2d = jnp.concatenate(
        [edge_index[1], jnp.full((pad,), N, jnp.int32)]).reshape(EP // BI, BI)
    ewt = jnp.concatenate(
        [edge_w.T, jnp.zeros((2, pad), jnp.float32)], axis=1)
    ew4t = jnp.concatenate([ewt[:, :EPH], ewt[:, EPH:]], axis=0)
    xcat = jnp.concatenate([x[:, :DH], x[:, DH:]], axis=0)
    xcat = (xcat.reshape(2 * N, 2, 2, 16).transpose(0, 1, 3, 2)
            .reshape(2 * N, DH).astype(jnp.bfloat16))
    z2 = jnp.zeros((2, DH), jnp.float32)
    w1p = jnp.concatenate(
        [jnp.concatenate([W1, z2], axis=1),
         jnp.concatenate([z2, W1], axis=1)], axis=0)
    zd = jnp.zeros((DH, DH), jnp.float32)
    w2d = jnp.stack([
        jnp.concatenate(
            [jnp.concatenate([W2[:, c * DH:(c + 1) * DH], zd], axis=1),
             jnp.concatenate([zd, W2[:, c * DH:(c + 1) * DH]], axis=1)],
            axis=0)
        for c in range(NC)])
    pe1 = _pe_call(ew4t, w1p, w2d.astype(jnp.bfloat16))
    acc, cntp = _sc_call(xcat, src2d, dst2d, pe1)
    return _fin_call(acc, cntp.reshape(NC * NS, NP).T, x, W, b, gamma, beta)


# speedup vs baseline: 1.6397x; 1.1071x over previous
import jax
import jax.numpy as jnp
from jax import lax
from jax.experimental import pallas as pl
from jax.experimental.pallas import tpu as pltpu
from jax.experimental.pallas import tpu_sc as plsc

N = 10000
E = 320000
D = 128
DH = D // 2

NC, NS = 2, 16
EP = 327680
NP = 10240
BI = 128
RW = EP // BI // NS
SCH = 128
RPS = SCH // BI
NCH = RW // RPS
RPT = NP // NS


EPH = EP // 2
BEH = 8192


def _pe_body(ew_ref, w1_ref, w2_ref, out_ref):
    hp = lax.dot_general(ew_ref[...], w1_ref[...],
                         (((0,), (0,)), ((), ())),
                         preferred_element_type=jnp.float32)
    hp = jnp.maximum(hp, 0.0).astype(jnp.bfloat16)
    pe = jnp.dot(hp, w2_ref[0], preferred_element_type=jnp.float32) + 1.0
    rb = pe.astype(jnp.bfloat16).astype(jnp.float32)
    u = lax.bitcast_convert_type(rb, jnp.uint32).reshape(BEH // 128, 2,
                                                         64, D)
    packed = u[:, 0] | (u[:, 1] >> 16)
    out_ref[0] = lax.bitcast_convert_type(
        packed, jnp.int32).reshape(BEH // 2, D)


def _pe_call(ew4t, w1p, w2d):
    return pl.pallas_call(
        _pe_body,
        grid=(EPH // BEH, NC),
        in_specs=[
            pl.BlockSpec((4, BEH), lambda i, h: (0, i)),
            pl.BlockSpec((4, D), lambda i, h: (0, 0)),
            pl.BlockSpec((1, D, D), lambda i, h: (h, 0, 0)),
        ],
        out_specs=pl.BlockSpec((1, BEH // 2, D), lambda i, h: (h, i, 0)),
        out_shape=jax.ShapeDtypeStruct((NC, EPH // 2, D), jnp.int32),
    )(ew4t, w1p, w2d)


BX = 2000


def _xp_body(x_ref, p_ref, out_ref):
    out_ref[...] = jnp.dot(x_ref[...], p_ref[...],
                           preferred_element_type=jnp.float32
                           ).astype(jnp.bfloat16)


def _xp_call(x, pmat):
    return pl.pallas_call(
        _xp_body,
        grid=(N // BX,),
        in_specs=[
            pl.BlockSpec((BX, D), lambda i: (i, 0)),
            pl.BlockSpec((D, D), lambda i: (0, 0)),
        ],
        out_specs=pl.BlockSpec((BX, D), lambda i: (i, 0)),
        out_shape=jax.ShapeDtypeStruct((N, D), jnp.bfloat16),
    )(x, pmat)


def _sc_body(x_hbm, src_hbm, dst_hbm, pe_hbm, acc_hbm, cnt_hbm,
             src_a, src_b, dst_v, pe_a, pe_b, xr_a, xr_b, ms_a, ms_b,
             cnt_v, acc_sh, semg_a, semg_b, semi_a, semi_b, sems_a, sems_b):
    c = lax.axis_index("c")
    s = lax.axis_index("s")

    zeros16 = jnp.zeros((16,), jnp.float32)
    ones16 = jnp.ones((16,), jnp.float32)

    @pl.loop(0, NP // 16)
    def _(i):
        cnt_v[pl.ds(i * 16, 16)] = zeros16

    @pl.loop(0, SCH)
    def _(r):
        for g in range(DH // 16):
            ms_a[r, pl.ds(g * 16, 16)] = zeros16

    for q in range(RPT // SCH):
        pltpu.sync_copy(ms_a, acc_sh.at[pl.ds(s * RPT + q * SCH, SCH)])

    pltpu.sync_copy(dst_hbm.at[pl.ds(s * RW, RW)], dst_v)
    plsc.subcore_barrier()

    shalf = s // 8
    coff = shalf * DH

    def idx_fire(k, src_buf, sem):
        pltpu.async_copy(src_hbm.at[pl.ds(s * RW + k, 1)], src_buf, sem)

    def idx_wait_gidx(src_buf, sem):
        pltpu.make_async_copy(src_hbm.at[pl.ds(0, 1)], src_buf, sem).wait()
        for g in range(BI // 16):
            sl = pl.ds(g * 16, 16)
            src_buf[0, sl] = src_buf[0, sl] + src_buf[0, sl] + c

    def fire(k, src_buf, pe_buf, xr_buf, sem):
        pltpu.async_copy(x_hbm.at[src_buf.at[0]], xr_buf, sem)
        prow = ((s * RW + k) * BI - shalf * EPH) // 2
        pltpu.async_copy(
            pe_hbm.at[c, pl.ds(prow, SCH // 2), pl.ds(coff, DH)],
            pe_buf, sem)

    def drain(src_buf, pe_buf, xr_buf, sem):
        pltpu.make_async_copy(x_hbm.at[src_buf.at[0]], xr_buf, sem).wait()
        pltpu.make_async_copy(
            pe_hbm.at[c, pl.ds(0, SCH // 2), pl.ds(0, DH)],
            pe_buf, sem).wait()

    def scatter_wait(ms_buf, sem):
        pltpu.make_async_copy(ms_buf, acc_sh.at[dst_v.at[0]], sem).wait()

    def chunk_step(kk, k, src_buf, pe_buf, xr_buf, ms_buf, semg, semi, sems):
        drain(src_buf, pe_buf, xr_buf, semg)

        @pl.when(k + 2 < NCH)
        def _():
            idx_fire(k + 2, src_buf, semi)

        @pl.when(kk > 0)
        def _():
            scatter_wait(ms_buf, sems)

        hmask = jnp.int32(-65536)

        @plsc.parallel_loop(0, SCH // 2, 1, unroll=2)
        def _(rp):
            for gg in range(DH // 32):
                w0 = pe_buf[rp, pl.ds(gg * 32, 16)]
                w1 = pe_buf[rp, pl.ds(gg * 32 + 16, 16)]
                xa, xb = plsc.unpack(xr_buf[rp, pl.ds(gg * 32, 32)],
                                     format=plsc.PackFormat.INTERLEAVED)
                ms_buf[rp, pl.ds(gg * 32, 16)] = \
                    xa * plsc.bitcast(w0 & hmask, jnp.float32)
                ms_buf[rp, pl.ds(gg * 32 + 16, 16)] = \
                    xb * plsc.bitcast(w1 & hmask, jnp.float32)
                r2 = rp + SCH // 2
                xc, xd = plsc.unpack(xr_buf[r2, pl.ds(gg * 32, 32)],
                                     format=plsc.PackFormat.INTERLEAVED)
                ms_buf[r2, pl.ds(gg * 32, 16)] = \
                    xc * plsc.bitcast(w0 << 16, jnp.float32)
                ms_buf[r2, pl.ds(gg * 32 + 16, 16)] = \
                    xd * plsc.bitcast(w1 << 16, jnp.float32)

        pltpu.async_copy(ms_buf, acc_sh.at[dst_v.at[k]], sems, add=True)

        @pl.when((c == 0) == (k < NCH // 2))
        def _():
            for g in range(BI // 16):
                idx16 = dst_v[k, pl.ds(g * 16, 16)]
                plsc.addupdate_scatter(cnt_v, [idx16], ones16)

        @pl.when(k + 2 < NCH)
        def _():
            idx_wait_gidx(src_buf, semi)
            fire(k + 2, src_buf, pe_buf, xr_buf, semg)

    pltpu.sync_copy(src_hbm.at[pl.ds(s * RW, 1)], src_a)
    pltpu.sync_copy(src_hbm.at[pl.ds(s * RW + 1, 1)], src_b)
    for g in range(BI // 16):
        sl = pl.ds(g * 16, 16)
        src_a[0, sl] = src_a[0, sl] + src_a[0, sl] + c
        src_b[0, sl] = src_b[0, sl] + src_b[0, sl] + c
    fire(0, src_a, pe_a, xr_a, semg_a)
    fire(1, src_b, pe_b, xr_b, semg_b)

    @pl.loop(0, NCH // 2)
    def _(kk):
        ka = 2 * kk
        chunk_step(kk, ka, src_a, pe_a, xr_a, ms_a, semg_a, semi_a, sems_a)
        chunk_step(kk, ka + 1, src_b, pe_b, xr_b, ms_b, semg_b, semi_b,
                   sems_b)

    scatter_wait(ms_a, sems_a)
    scatter_wait(ms_b, sems_b)
    plsc.subcore_barrier()

    pltpu.sync_copy(acc_sh.at[pl.ds(s * RPT, RPT)],
                    acc_hbm.at[c, pl.ds(s * RPT, RPT)])
    pltpu.sync_copy(cnt_v, cnt_hbm.at[pl.ds((c * NS + s) * NP, NP)])


_sc_call = pl.kernel(
    _sc_body,
    out_type=[
        jax.ShapeDtypeStruct((NC, NP, DH), jnp.float32),
        jax.ShapeDtypeStruct((NC * NS * NP,), jnp.float32),
    ],
    mesh=plsc.VectorSubcoreMesh(core_axis_name="c", subcore_axis_name="s"),
    compiler_params=pltpu.CompilerParams(needs_layout_passes=False,
                                         use_tc_tiling_on_sc=False),
    scratch_types=[
        pltpu.VMEM((1, BI), jnp.int32),
        pltpu.VMEM((1, BI), jnp.int32),
        pltpu.VMEM((RW, BI), jnp.int32),
        pltpu.VMEM((SCH // 2, DH), jnp.int32),
        pltpu.VMEM((SCH // 2, DH), jnp.int32),
        pltpu.VMEM((SCH, DH), jnp.bfloat16),
        pltpu.VMEM((SCH, DH), jnp.bfloat16),
        pltpu.VMEM((SCH, DH), jnp.float32),
        pltpu.VMEM((SCH, DH), jnp.float32),
        pltpu.VMEM((NP,), jnp.float32),
        pltpu.VMEM_SHARED((NP, DH), jnp.float32),
        pltpu.SemaphoreType.DMA,
        pltpu.SemaphoreType.DMA,
        pltpu.SemaphoreType.DMA,
        pltpu.SemaphoreType.DMA,
        pltpu.SemaphoreType.DMA,
        pltpu.SemaphoreType.DMA,
    ],
)


BN = 2000
NB = N // BN


def _fin_body(acc_ref, cnt_ref, x_ref, w_ref, b_ref, g_ref, be_ref,
              out_ref, t_sc, s1_sc, s2_sc):
    p = pl.program_id(0)
    i = pl.program_id(1)

    @pl.when(p == 0)
    def _():
        xb = x_ref[...]
        ssum = jnp.concatenate([acc_ref[0], acc_ref[1]], axis=1) + xb
        cnt = jnp.sum(cnt_ref[...], axis=1) + 1.0
        mean = ssum / cnt[:, None]
        wt = w_ref[...]
        t_pre = (
            jnp.dot(xb, wt[:D], preferred_element_type=jnp.float32)
            + jnp.dot(mean, wt[D:], preferred_element_type=jnp.float32)
            + b_ref[...][None, :]
        )
        nrm = jnp.sqrt(jnp.sum(t_pre * t_pre, axis=1, keepdims=True))
        t = t_pre / jnp.maximum(nrm, 1e-12)
        t_sc[pl.ds(i * BN, BN), :] = t

        @pl.when(i == 0)
        def _():
            s1_sc[...] = jnp.zeros_like(s1_sc)
            s2_sc[...] = jnp.zeros_like(s2_sc)

        s1_sc[...] += jnp.sum(t, axis=0, keepdims=True)
        s2_sc[...] += jnp.sum(t * t, axis=0, keepdims=True)

    @pl.when(p == 1)
    def _():
        t = t_sc[pl.ds(i * BN, BN), :]
        mu = s1_sc[...] / N
        var = s2_sc[...] / N - mu * mu
        y = (t - mu) * lax.rsqrt(var + 1e-5) * g_ref[...][None, :] \
            + be_ref[...][None, :]
        out_ref[...] = jnp.maximum(y + x_ref[...], 0.0)


def _fin_call(acc, cntp, x, W, b, gamma, beta):
    return pl.pallas_call(
        _fin_body,
        grid=(2, NB),
        in_specs=[
            pl.BlockSpec((NC, BN, DH), lambda p, i: (0, i, 0)),
            pl.BlockSpec((BN, NC * NS), lambda p, i: (i, 0)),
            pl.BlockSpec((BN, D), lambda p, i: (i, 0)),
            pl.BlockSpec((2 * D, D), lambda p, i: (0, 0)),
            pl.BlockSpec((D,), lambda p, i: (0,)),
            pl.BlockSpec((D,), lambda p, i: (0,)),
            pl.BlockSpec((D,), lambda p, i: (0,)),
        ],
        out_specs=pl.BlockSpec((BN, D), lambda p, i: (i, 0)),
        out_shape=jax.ShapeDtypeStruct((N, D), jnp.float32),
        scratch_shapes=[
            pltpu.VMEM((N, D), jnp.float32),
            pltpu.VMEM((1, D), jnp.float32),
            pltpu.VMEM((1, D), jnp.float32),
        ],
    )(acc, cntp, x, W, b, gamma, beta)


def kernel(x, edge_index, edge_w, W1, W2, W, b, gamma, beta):
    pad = EP - E
    src2d = jnp.concatenate(
        [edge_index[0], jnp.zeros((pad,), jnp.int32)]).reshape(EP // BI, BI)
    dst2d = jnp.concatenate(
        [edge_index[1],
         N + jnp.arange(pad, dtype=jnp.int32) % (NP - N)]
    ).reshape(EP // BI, BI)
    ewt = jnp.concatenate(
        [edge_w.T, jnp.zeros((2, pad), jnp.float32)], axis=1)
    ew4t = jnp.concatenate([ewt[:, :EPH], ewt[:, EPH:]], axis=0)
    qd = jnp.arange(D)
    u = qd % DH
    src_feat = DH * (qd // DH) + 32 * (u // 32) + (u % 32) // 2 + 16 * (u % 2)
    pmat = (jnp.arange(D)[:, None] == src_feat[None, :]).astype(jnp.float32)
    xcat = _xp_call(x, pmat).reshape(2 * N, DH)
    z2 = jnp.zeros((2, DH), jnp.float32)
    w1p = jnp.concatenate(
        [jnp.concatenate([W1, z2], axis=1),
         jnp.concatenate([z2, W1], axis=1)], axis=0)
    zd = jnp.zeros((DH, DH), jnp.float32)
    w2d = jnp.stack([
        jnp.concatenate(
            [jnp.concatenate([W2[:, c * DH:(c + 1) * DH], zd], axis=1),
             jnp.concatenate([zd, W2[:, c * DH:(c + 1) * DH]], axis=1)],
            axis=0)
        for c in range(NC)])
    pe1 = _pe_call(ew4t, w1p, w2d.astype(jnp.bfloat16))
    acc, cntp = _sc_call(xcat, src2d, dst2d, pe1)
    return _fin_call(acc, cntp.reshape(NC * NS, NP).T, x, W, b, gamma, beta)
